# baseline (speedup 1.0000x reference)
import functools

import jax
import jax.numpy as jnp
from jax import lax
from jax.experimental import pallas as pl
from jax.experimental.pallas import tpu as pltpu
from jax.experimental.pallas import tpu_sc as plsc

_L = 16
_BATCH = 32
_NW = 32


def _sc_body(ids_hbm, tok_hbm, emb_hbm, vis_hbm, out_hbm,
             ids_v, tok_v, didx, sidx, rbuf, sem):
    n = ids_hbm.shape[0]
    rw = n // _NW
    wid = lax.axis_index("s") * 2 + lax.axis_index("c")
    base = wid * rw

    pltpu.sync_copy(ids_hbm, ids_v)
    pltpu.sync_copy(tok_hbm, tok_v)
    tokv = tok_v[...]

    iota = lax.iota(jnp.int32, _L)

    def mask_at(c):
        return ids_v[pl.ds(c * _L, _L)] == tokv

    def pref_step(c, acc):
        return acc + jnp.sum(mask_at(c).astype(jnp.int32))
    prefix = lax.fori_loop(0, base // _L, pref_step, jnp.int32(0))

    big = jnp.int32(n)

    def ff_step(c, carry):
        fm, fu = carry
        m = mask_at(c)
        rows = base + c * _L + iota
        cm = jnp.min(jnp.where(m, rows, big))
        cu = jnp.min(jnp.where(m, big, rows))
        return jnp.minimum(fm, cm), jnp.minimum(fu, cu)
    c0 = base // _L
    fm, fu = lax.fori_loop(c0, c0 + rw // _L, ff_step, (big, big))
    pad_vsrc = prefix

    nchunks = _BATCH // _L

    def batch_step(bt, off):
        didx[pl.ds(0, _L)] = jnp.full((_L,), fm, jnp.int32)
        sidx[pl.ds(0, _L)] = jnp.full((_L,), pad_vsrc, jnp.int32)
        for q in range(1, nchunks):
            didx[pl.ds(q * _L, _L)] = jnp.full((_L,), fm, jnp.int32)
            sidx[pl.ds(q * _L, _L)] = jnp.full((_L,), pad_vsrc, jnp.int32)

        def chunk_step(q, carry):
            off_q, cnt = carry
            c = c0 + bt * nchunks + q
            m = mask_at(c)
            ks = jnp.sum(m.astype(jnp.int32))
            rows = base + (bt * nchunks + q) * _L + iota
            pos = off_q + plsc.cumsum(m.astype(jnp.int32)) - 1
            plsc.store_compressed(didx.at[pl.ds(cnt, _L)], rows, mask=m)
            plsc.store_compressed(sidx.at[pl.ds(cnt, _L)], pos, mask=m)
            return off_q + ks, cnt + ks
        off_end, mcnt = lax.fori_loop(0, nchunks, chunk_step,
                                      (off, jnp.int32(0)))

        @pl.when(mcnt > 0)
        def _():
            pltpu.async_copy(vis_hbm.at[sidx], rbuf, sem).wait()
            pltpu.async_copy(rbuf, out_hbm.at[didx], sem).wait()

        didx[pl.ds(0, _L)] = jnp.full((_L,), fu, jnp.int32)
        for q in range(1, nchunks):
            didx[pl.ds(q * _L, _L)] = jnp.full((_L,), fu, jnp.int32)

        def chunk_step_u(q, cnt):
            c = c0 + bt * nchunks + q
            m = mask_at(c)
            rows = base + (bt * nchunks + q) * _L + iota
            plsc.store_compressed(didx.at[pl.ds(cnt, _L)], rows,
                                  mask=jnp.logical_not(m))
            return cnt + (_L - jnp.sum(m.astype(jnp.int32)))
        ucnt = lax.fori_loop(0, nchunks, chunk_step_u, jnp.int32(0))

        @pl.when(ucnt > 0)
        def _():
            pltpu.async_copy(emb_hbm.at[didx], rbuf, sem).wait()
            pltpu.async_copy(rbuf, out_hbm.at[didx], sem).wait()

        return off_end

    lax.fori_loop(0, rw // _BATCH, batch_step, prefix)


def sc_kernel(input_ids, inputs_embeds, vision_embeddings, image_token_id):
    b, s, d = inputs_embeds.shape
    n = b * s
    ids = input_ids.reshape(n)
    tok = jnp.full((_L,), image_token_id, jnp.int32)
    embeds = inputs_embeds.reshape(n, d)
    vis = vision_embeddings.reshape(n, d)

    mesh = plsc.VectorSubcoreMesh(core_axis_name="c", subcore_axis_name="s")
    run = functools.partial(
        pl.kernel,
        mesh=mesh,
        out_type=jax.ShapeDtypeStruct((n, d), jnp.float32),
        compiler_params=pltpu.CompilerParams(needs_layout_passes=False),
        scratch_types=[
            pltpu.VMEM((n,), jnp.int32),
            pltpu.VMEM((_L,), jnp.int32),
            pltpu.VMEM((_BATCH,), jnp.int32),
            pltpu.VMEM((_BATCH,), jnp.int32),
            pltpu.VMEM((_BATCH, d), jnp.float32),
            pltpu.SemaphoreType.DMA,
        ],
    )(_sc_body)
    out = run(ids, tok, embeds, vis)
    return out.reshape(b, s, d)


kernel = sc_kernel


# speedup vs baseline: 1.0722x; 1.0722x over previous
import functools

import jax
import jax.numpy as jnp
from jax import lax
from jax.experimental import pallas as pl
from jax.experimental.pallas import tpu as pltpu
from jax.experimental.pallas import tpu_sc as plsc

_L = 16
_B = 32
_NW = 32
_NC = _B // _L


def _sc_body(ids_hbm, tok_hbm, emb_hbm, vis_hbm, out_hbm,
             idsw, tok_v, didx, sidx, eidx, rbuf, ebuf,
             gsem, ssem, esem, offc, mcs):
    n = ids_hbm.shape[0]
    rw = n // _NW
    wid = lax.axis_index("s") * 2 + lax.axis_index("c")
    base = wid * rw
    nb = rw // _B

    pltpu.sync_copy(tok_hbm, tok_v)
    tokv = tok_v[...]
    iota = lax.iota(jnp.int32, _L)

    def wprefix(w, acc):
        pltpu.sync_copy(ids_hbm.at[pl.ds(w * rw, rw)], idsw)

        def stp(c, a):
            m = idsw[pl.ds(c * _L, _L)] == tokv
            return a + plsc.all_reduce_population_count(m)[0]
        return lax.fori_loop(0, rw // _L, stp, acc)
    prefix = lax.fori_loop(0, wid, wprefix, jnp.int32(0))

    pltpu.sync_copy(ids_hbm.at[pl.ds(base, rw)], idsw)

    def mask_at(c):
        return idsw[pl.ds(c * _L, _L)] == tokv

    big = jnp.int32(n - 1)

    def ff_step(c, carry):
        fm, fu = carry
        m = mask_at(c)
        km = plsc.all_reduce_population_count(m)[0]
        f1 = plsc.all_reduce_ffs(m)[0]
        f0 = plsc.all_reduce_ffs(jnp.logical_not(m))[0]
        row0 = base + c * _L
        fm = jnp.where(jnp.logical_and(fm == big, km > 0), row0 + f1, fm)
        fu = jnp.where(jnp.logical_and(fu == big, km < _L), row0 + f0, fu)
        return fm, fu
    fm, fu = lax.fori_loop(0, rw // _L, ff_step, (big, big))

    def build(bt, slot):
        off = offc[0]

        def chunk(q, cnt):
            m = mask_at(bt * _NC + q)
            ks = plsc.all_reduce_population_count(m)[0]
            rows = base + (bt * _NC + q) * _L + iota
            plsc.store_compressed(didx.at[slot].at[pl.ds(cnt, _L)], rows,
                                  mask=m)
            return cnt + ks
        mcnt = lax.fori_loop(0, _NC, chunk, jnp.int32(0))
        for q in range(_NC):
            lane = q * _L + iota
            tail = lane >= mcnt
            cur = didx[slot, pl.ds(q * _L, _L)]
            didx[slot, pl.ds(q * _L, _L)] = jnp.where(tail, fm, cur)
            sidx[slot, pl.ds(q * _L, _L)] = jnp.where(tail, prefix, off + lane)
        offc[0] = off + mcnt
        mcs[slot] = mcnt

    def gather_start(slot):
        pltpu.make_async_copy(vis_hbm.at[sidx.at[slot]], rbuf.at[slot],
                              gsem.at[slot]).start()

    def gather_wait(slot):
        pltpu.make_async_copy(vis_hbm.at[sidx.at[slot]], rbuf.at[slot],
                              gsem.at[slot]).wait()

    def scatter_start(slot):
        pltpu.make_async_copy(rbuf.at[slot], out_hbm.at[didx.at[slot]],
                              ssem.at[slot]).start()

    def scatter_wait(slot):
        pltpu.make_async_copy(rbuf.at[slot], out_hbm.at[didx.at[slot]],
                              ssem.at[slot]).wait()

    def fire(bt, slot):
        build(bt, slot)

        @pl.when(mcs[slot] > 0)
        def _():
            gather_start(slot)

    def embeds_batch(bt):
        def chunk(q, cnt):
            m = mask_at(bt * _NC + q)
            ks = plsc.all_reduce_population_count(m)[0]
            rows = base + (bt * _NC + q) * _L + iota
            plsc.store_compressed(eidx.at[pl.ds(cnt, _L)], rows,
                                  mask=jnp.logical_not(m))
            return cnt + (_L - ks)
        ucnt = lax.fori_loop(0, _NC, chunk, jnp.int32(0))

        @pl.when(ucnt > 0)
        def _():
            for q in range(_NC):
                lane = q * _L + iota
                tail = lane >= ucnt
                cur = eidx[pl.ds(q * _L, _L)]
                eidx[pl.ds(q * _L, _L)] = jnp.where(tail, fu, cur)
            pltpu.make_async_copy(emb_hbm.at[eidx], ebuf, esem).start()
            pltpu.make_async_copy(emb_hbm.at[eidx], ebuf, esem).wait()
            pltpu.make_async_copy(ebuf, out_hbm.at[eidx], esem).start()
            pltpu.make_async_copy(ebuf, out_hbm.at[eidx], esem).wait()

    offc[0] = prefix
    fire(jnp.int32(0), 0)

    def body(bt2, carry):
        for b in range(2):
            bt = bt2 * 2 + b
            slot = b
            nslot = 1 - b

            @pl.when(bt + 1 < nb)
            def _():
                @pl.when(jnp.logical_and(bt >= 1, mcs[nslot] > 0))
                def _():
                    scatter_wait(nslot)
                fire(bt + 1, nslot)

            @pl.when(mcs[slot] > 0)
            def _():
                gather_wait(slot)
                scatter_start(slot)

            embeds_batch(bt)
        return carry

    lax.fori_loop(0, nb // 2, body, jnp.int32(0))

    for slot in range(2):
        @pl.when(mcs[slot] > 0)
        def _():
            scatter_wait(slot)


def sc_kernel(input_ids, inputs_embeds, vision_embeddings, image_token_id):
    b, s, d = inputs_embeds.shape
    n = b * s
    ids = input_ids.reshape(n)
    tok = jnp.full((_L,), image_token_id, jnp.int32)
    embeds = inputs_embeds.reshape(n, d)
    vis = vision_embeddings.reshape(n, d)

    mesh = plsc.VectorSubcoreMesh(core_axis_name="c", subcore_axis_name="s")
    run = functools.partial(
        pl.kernel,
        mesh=mesh,
        out_type=jax.ShapeDtypeStruct((n, d), jnp.float32),
        compiler_params=pltpu.CompilerParams(needs_layout_passes=False),
        scratch_types=[
            pltpu.VMEM((n // _NW,), jnp.int32),
            pltpu.VMEM((_L,), jnp.int32),
            pltpu.VMEM((2, _B), jnp.int32),
            pltpu.VMEM((2, _B), jnp.int32),
            pltpu.VMEM((_B,), jnp.int32),
            pltpu.VMEM((2, _B, d), jnp.float32),
            pltpu.VMEM((_B, d), jnp.float32),
            pltpu.SemaphoreType.DMA((2,)),
            pltpu.SemaphoreType.DMA((2,)),
            pltpu.SemaphoreType.DMA,
            pltpu.SMEM((1,), jnp.int32),
            pltpu.SMEM((2,), jnp.int32),
        ],
    )(_sc_body)
    out = run(ids, tok, embeds, vis)
    return out.reshape(b, s, d)


kernel = sc_kernel
